# Initial kernel scaffold; baseline (speedup 1.0000x reference)
import jax
import jax.numpy as jnp
from jax.experimental import pallas as pl


def kernel(node_feats, edge_index, batch, temperature, t, Wl0, bl0, Wr0, Wl1, bl1, Wr1, Wl2, bl2, Wr2, Wv, bv, W1, b1, W2, b2):
    raise NotImplementedError("write your pallas kernel here")


# speedup vs baseline: 6.1899x; 6.1899x over previous
import jax
import jax.numpy as jnp
from jax import lax
from jax.experimental import pallas as pl
from jax.experimental.pallas import tpu as pltpu
from jax.experimental.pallas import tpu_sc as plsc

NN = 50000
HID = 64
F = 32
EE = 800000
CH = 128
NSUB = 16
NCORE = 2
EPAD = 802816
MAIN_CHUNKS = EPAD // (NSUB * CH)
HALF_CHUNKS = EPAD // (2 * NSUB * CH)
EHALF = EPAD // 2
NPAD = 51200
ZROWS = NPAD // NSUB
NOUT = 50048
OROWS = NOUT // NSUB
SW = 16
BN = 2000
GRID = NN // BN


def _sc_scalar_body(xs, srcp, dstp, zer, out, src_v, dst_v, rows_v, acc, sem):
    c = lax.axis_index("c")
    s = lax.axis_index("s")
    pltpu.sync_copy(zer, acc.at[pl.ds(s * ZROWS, ZROWS)])
    plsc.subcore_barrier()
    base = c * EHALF + s * (HALF_CHUNKS * CH)

    def body(i, carry):
        off = base + i * CH
        pltpu.sync_copy(srcp.at[pl.ds(off, CH)], src_v)
        pltpu.sync_copy(dstp.at[pl.ds(off, CH)], dst_v)
        pltpu.async_copy(xs.at[src_v], rows_v, sem).wait()
        pltpu.sync_copy(rows_v, acc.at[dst_v], add=True)
        return carry

    lax.fori_loop(0, HALF_CHUNKS, body, 0)
    plsc.subcore_barrier()
    pltpu.sync_copy(acc.at[pl.ds(s * OROWS, OROWS)],
                    out.at[c, pl.ds(s * OROWS, OROWS)])


def _sc_agg_body(h2, srcp, dstp, zer, out, src_v, dst_v, rows_v, acc, sem):
    c = lax.axis_index("c")
    s = lax.axis_index("s")
    pltpu.sync_copy(zer, acc.at[pl.ds(s * ZROWS, ZROWS)])
    plsc.subcore_barrier()
    base = s * (MAIN_CHUNKS * CH)

    def body(i, carry):
        off = base + i * CH
        pltpu.sync_copy(srcp.at[pl.ds(off, CH)], src_v)
        pltpu.sync_copy(dstp.at[pl.ds(off, CH)], dst_v)
        pltpu.async_copy(h2.at[c].at[src_v], rows_v, sem).wait()
        pltpu.sync_copy(rows_v, acc.at[dst_v], add=True)
        return carry

    lax.fori_loop(0, MAIN_CHUNKS, body, 0)
    plsc.subcore_barrier()
    pltpu.sync_copy(acc.at[pl.ds(s * OROWS, OROWS)],
                    out.at[c, pl.ds(s * OROWS, OROWS)])


def _sc_mesh():
    return plsc.VectorSubcoreMesh(core_axis_name="c", subcore_axis_name="s",
                                  num_cores=NCORE, num_subcores=NSUB)


def _scalar_pass(xs, srcp, dstp, zer):
    return pl.kernel(
        _sc_scalar_body,
        out_type=jax.ShapeDtypeStruct((NCORE, NOUT, SW), jnp.float32),
        mesh=_sc_mesh(),
        scratch_types=[
            pltpu.VMEM((CH,), jnp.int32),
            pltpu.VMEM((CH,), jnp.int32),
            pltpu.VMEM((CH, SW), jnp.float32),
            pltpu.VMEM_SHARED((NPAD, SW), jnp.float32),
            pltpu.SemaphoreType.DMA,
        ],
        compiler_params=pltpu.CompilerParams(use_tc_tiling_on_sc=False),
    )(xs, srcp, dstp, zer)


def _agg_pass(h2, srcp, dstp, zer):
    return pl.kernel(
        _sc_agg_body,
        out_type=jax.ShapeDtypeStruct((NCORE, NOUT, F), jnp.float32),
        mesh=_sc_mesh(),
        scratch_types=[
            pltpu.VMEM((CH,), jnp.int32),
            pltpu.VMEM((CH,), jnp.int32),
            pltpu.VMEM((CH, F), jnp.float32),
            pltpu.VMEM_SHARED((NPAD, F), jnp.float32),
            pltpu.SemaphoreType.DMA,
        ],
        compiler_params=pltpu.CompilerParams(use_tc_tiling_on_sc=False),
    )(h2, srcp, dstp, zer)


def _tc_stage0_body(x, p, wl, bl, wr, h_out, dinv_out):
    pb = p[...]
    deg = pb[0, :, 0:1] + pb[1, :, 0:1]
    di = 1.0 / jnp.maximum(deg, 1.0)
    a0 = (pb[0, :, 1:2] + pb[1, :, 1:2]) * di
    h1 = jnp.maximum(a0 * wl[...] + bl[...] + x[...] * wr[...], 0.0)
    dinv_out[...] = di
    h_out[0] = h1[:, :F]
    h_out[1] = h1[:, F:]


def _tc_mid_body(h, agg, dinv, wlT, bl, wrT, out):
    hb = jnp.concatenate([h[0], h[1]], axis=1)
    ab = jnp.concatenate([agg[0], agg[1]], axis=1) * dinv[...]
    hn = jnp.maximum(
        jnp.dot(ab, wlT[...], preferred_element_type=jnp.float32, precision=lax.Precision.HIGHEST) + bl[...]
        + jnp.dot(hb, wrT[...], preferred_element_type=jnp.float32, precision=lax.Precision.HIGHEST), 0.0)
    out[0] = hn[:, :F]
    out[1] = hn[:, F:]


def _tc_final_body(h, agg, dinv, wlT, bl, wrT, wvT, bv, w1T, b1, w2T, b2, ex,
                   out, acc):
    i = pl.program_id(0)

    @pl.when(i == 0)
    def _():
        acc[...] = jnp.zeros_like(acc)

    hb = jnp.concatenate([h[0], h[1]], axis=1)
    ab = jnp.concatenate([agg[0], agg[1]], axis=1) * dinv[...]
    h3 = jnp.maximum(
        jnp.dot(ab, wlT[...], preferred_element_type=jnp.float32, precision=lax.Precision.HIGHEST) + bl[...]
        + jnp.dot(hb, wrT[...], preferred_element_type=jnp.float32, precision=lax.Precision.HIGHEST), 0.0)
    acc[...] += jnp.sum(h3, axis=0, keepdims=True)

    @pl.when(i == GRID - 1)
    def _():
        pooled = acc[...] * (1.0 / NN)
        z = jnp.dot(pooled, wvT[...], preferred_element_type=jnp.float32, precision=lax.Precision.HIGHEST) + bv[...]
        feat = jnp.concatenate([z, ex[...]], axis=1)
        hm = jnp.maximum(
            jnp.dot(feat, w1T[...], preferred_element_type=jnp.float32, precision=lax.Precision.HIGHEST) + b1[...], 0.0)
        oo = jnp.dot(hm, w2T[...], preferred_element_type=jnp.float32, precision=lax.Precision.HIGHEST) + b2[...]
        out[...] = jnp.maximum(oo, 0.0) + jnp.log1p(jnp.exp(-jnp.abs(oo)))


def _full(i):
    return (0,) * i


def kernel(node_feats, edge_index, batch, temperature, t,
           Wl0, bl0, Wr0, Wl1, bl1, Wr1, Wl2, bl2, Wr2,
           Wv, bv, W1, b1, W2, b2):
    f32 = jnp.float32
    src = edge_index[0]
    dst = edge_index[1]
    pad = EPAD - EE
    srcp = jnp.concatenate([src, jnp.zeros((pad,), jnp.int32)])
    dstp = jnp.concatenate([dst, jnp.full((pad,), NN, jnp.int32)])
    x = node_feats.astype(f32)
    xs = jnp.concatenate([jnp.ones((NN, 1), f32), x, jnp.zeros((NN, SW - 2), f32)], axis=1)
    zer2 = jnp.zeros((ZROWS, SW), f32)
    zer32 = jnp.zeros((ZROWS, F), f32)

    part = _scalar_pass(xs, srcp, dstp, zer2)

    h, dinv = pl.pallas_call(
        _tc_stage0_body,
        grid=(GRID,),
        in_specs=[
            pl.BlockSpec((BN, 1), lambda i: (i, 0)),
            pl.BlockSpec((2, BN, SW), lambda i: (0, i, 0)),
            pl.BlockSpec((1, HID), lambda i: (0, 0)),
            pl.BlockSpec((1, HID), lambda i: (0, 0)),
            pl.BlockSpec((1, HID), lambda i: (0, 0)),
        ],
        out_specs=[
            pl.BlockSpec((2, BN, F), lambda i: (0, i, 0)),
            pl.BlockSpec((BN, 1), lambda i: (i, 0)),
        ],
        out_shape=[
            jax.ShapeDtypeStruct((2, NN, F), f32),
            jax.ShapeDtypeStruct((NN, 1), f32),
        ],
    )(x, part, Wl0.T, bl0.reshape(1, HID), Wr0.T)

    wmid = lambda Wl, bl, Wr: (Wl.T, bl.reshape(1, HID), Wr.T)

    agg1 = _agg_pass(h, srcp, dstp, zer32)
    h = pl.pallas_call(
        _tc_mid_body,
        grid=(GRID,),
        in_specs=[
            pl.BlockSpec((2, BN, F), lambda i: (0, i, 0)),
            pl.BlockSpec((2, BN, F), lambda i: (0, i, 0)),
            pl.BlockSpec((BN, 1), lambda i: (i, 0)),
            pl.BlockSpec((HID, HID), lambda i: (0, 0)),
            pl.BlockSpec((1, HID), lambda i: (0, 0)),
            pl.BlockSpec((HID, HID), lambda i: (0, 0)),
        ],
        out_specs=pl.BlockSpec((2, BN, F), lambda i: (0, i, 0)),
        out_shape=jax.ShapeDtypeStruct((2, NN, F), f32),
    )(h, agg1, dinv, *wmid(Wl1, bl1, Wr1))

    agg2 = _agg_pass(h, srcp, dstp, zer32)
    ex = jnp.array([[temperature, t]], dtype=f32)
    out = pl.pallas_call(
        _tc_final_body,
        grid=(GRID,),
        in_specs=[
            pl.BlockSpec((2, BN, F), lambda i: (0, i, 0)),
            pl.BlockSpec((2, BN, F), lambda i: (0, i, 0)),
            pl.BlockSpec((BN, 1), lambda i: (i, 0)),
            pl.BlockSpec((HID, HID), lambda i: (0, 0)),
            pl.BlockSpec((1, HID), lambda i: (0, 0)),
            pl.BlockSpec((HID, HID), lambda i: (0, 0)),
            pl.BlockSpec((HID, HID), lambda i: (0, 0)),
            pl.BlockSpec((1, HID), lambda i: (0, 0)),
            pl.BlockSpec((HID + 2, HID), lambda i: (0, 0)),
            pl.BlockSpec((1, HID), lambda i: (0, 0)),
            pl.BlockSpec((HID, 1), lambda i: (0, 0)),
            pl.BlockSpec((1, 1), lambda i: (0, 0)),
            pl.BlockSpec((1, 2), lambda i: (0, 0)),
        ],
        out_specs=pl.BlockSpec((1, 1), lambda i: (0, 0)),
        out_shape=jax.ShapeDtypeStruct((1, 1), f32),
        scratch_shapes=[pltpu.VMEM((1, HID), f32)],
    )(h, agg2, dinv, *wmid(Wl2, bl2, Wr2), Wv.T, bv.reshape(1, HID),
      W1.T, b1.reshape(1, HID), W2.T, b2.reshape(1, 1), ex)
    return out


# speedup vs baseline: 14.1227x; 2.2816x over previous
import jax
import jax.numpy as jnp
from jax import lax
from jax.experimental import pallas as pl
from jax.experimental.pallas import tpu as pltpu
from jax.experimental.pallas import tpu_sc as plsc

NN = 50000
HID = 64
F = 32
EE = 800000
CH = 128
NSUB = 16
NCORE = 2
EPAD = 802816
MAIN_CHUNKS = EPAD // (NSUB * CH)
HALF_CHUNKS = EPAD // (2 * NSUB * CH)
EHALF = EPAD // 2
NPAD = 51200
ZROWS = NPAD // NSUB
NOUT = 50048
OROWS = NOUT // NSUB
SW = 16
BN = 2000
GRID = NN // BN


def _make_sc_body(width, nbuf, chunks, scalar_mode):
    rounds = chunks // nbuf
    assert rounds % 2 == 0

    def body(table, srcp, dstp2, zer, out, sidx, didx, rows, acc, isem, gsem):
        c = lax.axis_index("c")
        s = lax.axis_index("s")
        if scalar_mode:
            chunk_base = c * (EPAD // (2 * CH)) + s * chunks
            tab = table
        else:
            chunk_base = s * chunks
            tab = table.at[c]
        pltpu.sync_copy(zer, acc.at[pl.ds(s * ZROWS, ZROWS)])
        plsc.subcore_barrier()

        def issue_idx(r, h):
            row0 = chunk_base + r * nbuf
            pltpu.async_copy(srcp.at[pl.ds(row0 * CH, nbuf * CH)],
                             sidx.at[h], isem.at[h])
            pltpu.async_copy(dstp2.at[pl.ds(row0, nbuf)],
                             didx.at[h], isem.at[h])

        def wait_idx(h):
            pltpu.make_async_copy(srcp.at[pl.ds(0, nbuf * CH)],
                                  sidx.at[h], isem.at[h]).wait()
            pltpu.make_async_copy(dstp2.at[pl.ds(0, nbuf)],
                                  didx.at[h], isem.at[h]).wait()

        def issue_gathers(h):
            for b in range(nbuf):
                pltpu.async_copy(tab.at[sidx.at[h, pl.ds(b * CH, CH)]],
                                 rows.at[h, b], gsem.at[h])

        def wait_gathers(h):
            for b in range(nbuf):
                pltpu.make_async_copy(tab.at[sidx.at[h, pl.ds(b * CH, CH)]],
                                      rows.at[h, b], gsem.at[h]).wait()

        def scatters(h):
            for b in range(nbuf):
                pltpu.sync_copy(rows.at[h, b], acc.at[didx.at[h, b]], add=True)

        def round_body(r, h, do_next, do_prefetch):
            if do_next:
                wait_idx(1 - h)
                issue_gathers(1 - h)
            wait_gathers(h)
            scatters(h)
            if do_prefetch:
                issue_idx(r + 2, h)

        issue_idx(0, 0)
        issue_idx(1, 1)
        wait_idx(0)
        issue_gathers(0)

        def pair(k2, carry):
            round_body(2 * k2, 0, True, True)
            round_body(2 * k2 + 1, 1, True, True)
            return carry

        lax.fori_loop(0, (rounds - 2) // 2, pair, 0)
        round_body(rounds - 2, 0, True, False)
        round_body(rounds - 1, 1, False, False)

        plsc.subcore_barrier()
        pltpu.sync_copy(acc.at[pl.ds(s * OROWS, OROWS)],
                        out.at[c, pl.ds(s * OROWS, OROWS)])

    return body, rounds


def _sc_mesh():
    return plsc.VectorSubcoreMesh(core_axis_name="c", subcore_axis_name="s",
                                  num_cores=NCORE, num_subcores=NSUB)


def _scalar_pass(xs, srcp, dstp2, zer):
    nbuf = 2
    body, _ = _make_sc_body(SW, nbuf, HALF_CHUNKS, True)
    return pl.kernel(
        body,
        out_type=jax.ShapeDtypeStruct((NCORE, NOUT, SW), jnp.float32),
        mesh=_sc_mesh(),
        scratch_types=[
            pltpu.VMEM((2, nbuf * CH), jnp.int32),
            pltpu.VMEM((2, nbuf, CH), jnp.int32),
            pltpu.VMEM((2, nbuf, CH, SW), jnp.float32),
            pltpu.VMEM_SHARED((NPAD, SW), jnp.float32),
            pltpu.SemaphoreType.DMA((2,)),
            pltpu.SemaphoreType.DMA((2,)),
        ],
        compiler_params=pltpu.CompilerParams(use_tc_tiling_on_sc=False),
    )(xs, srcp, dstp2, zer)


def _agg_pass(h2, srcp, dstp2, zer):
    nbuf = 2
    body, _ = _make_sc_body(F, nbuf, MAIN_CHUNKS, False)
    return pl.kernel(
        body,
        out_type=jax.ShapeDtypeStruct((NCORE, NOUT, F), jnp.float32),
        mesh=_sc_mesh(),
        scratch_types=[
            pltpu.VMEM((2, nbuf * CH), jnp.int32),
            pltpu.VMEM((2, nbuf, CH), jnp.int32),
            pltpu.VMEM((2, nbuf, CH, F), jnp.float32),
            pltpu.VMEM_SHARED((NPAD, F), jnp.float32),
            pltpu.SemaphoreType.DMA((2,)),
            pltpu.SemaphoreType.DMA((2,)),
        ],
        compiler_params=pltpu.CompilerParams(use_tc_tiling_on_sc=False),
    )(h2, srcp, dstp2, zer)


def _tc_stage0_body(x, p, wl, bl, wr, h_out, dinv_out):
    pb = p[...]
    deg = pb[0, :, 0:1] + pb[1, :, 0:1]
    di = 1.0 / jnp.maximum(deg, 1.0)
    a0 = (pb[0, :, 1:2] + pb[1, :, 1:2]) * di
    h1 = jnp.maximum(a0 * wl[...] + bl[...] + x[...] * wr[...], 0.0)
    dinv_out[...] = di
    h_out[0] = h1[:, :F]
    h_out[1] = h1[:, F:]


def _tc_mid_body(h, agg, dinv, wlT, bl, wrT, out):
    hb = jnp.concatenate([h[0], h[1]], axis=1)
    ab = jnp.concatenate([agg[0], agg[1]], axis=1) * dinv[...]
    hn = jnp.maximum(
        jnp.dot(ab, wlT[...], preferred_element_type=jnp.float32, precision=lax.Precision.HIGHEST) + bl[...]
        + jnp.dot(hb, wrT[...], preferred_element_type=jnp.float32, precision=lax.Precision.HIGHEST), 0.0)
    out[0] = hn[:, :F]
    out[1] = hn[:, F:]


def _tc_final_body(h, agg, dinv, wlT, bl, wrT, wvT, bv, w1T, b1, w2T, b2, ex,
                   out, acc):
    i = pl.program_id(0)

    @pl.when(i == 0)
    def _():
        acc[...] = jnp.zeros_like(acc)

    hb = jnp.concatenate([h[0], h[1]], axis=1)
    ab = jnp.concatenate([agg[0], agg[1]], axis=1) * dinv[...]
    h3 = jnp.maximum(
        jnp.dot(ab, wlT[...], preferred_element_type=jnp.float32, precision=lax.Precision.HIGHEST) + bl[...]
        + jnp.dot(hb, wrT[...], preferred_element_type=jnp.float32, precision=lax.Precision.HIGHEST), 0.0)
    acc[...] += jnp.sum(h3, axis=0, keepdims=True)

    @pl.when(i == GRID - 1)
    def _():
        pooled = acc[...] * (1.0 / NN)
        z = jnp.dot(pooled, wvT[...], preferred_element_type=jnp.float32, precision=lax.Precision.HIGHEST) + bv[...]
        feat = jnp.concatenate([z, ex[...]], axis=1)
        hm = jnp.maximum(
            jnp.dot(feat, w1T[...], preferred_element_type=jnp.float32, precision=lax.Precision.HIGHEST) + b1[...], 0.0)
        oo = jnp.dot(hm, w2T[...], preferred_element_type=jnp.float32, precision=lax.Precision.HIGHEST) + b2[...]
        out[...] = jnp.maximum(oo, 0.0) + jnp.log1p(jnp.exp(-jnp.abs(oo)))


def _full(i):
    return (0,) * i


def kernel(node_feats, edge_index, batch, temperature, t,
           Wl0, bl0, Wr0, Wl1, bl1, Wr1, Wl2, bl2, Wr2,
           Wv, bv, W1, b1, W2, b2):
    f32 = jnp.float32
    src = edge_index[0]
    dst = edge_index[1]
    pad = EPAD - EE
    srcp = jnp.concatenate([src, jnp.zeros((pad,), jnp.int32)])
    dstp = jnp.concatenate([dst, jnp.full((pad,), NN, jnp.int32)])
    x = node_feats.astype(f32)
    xs = jnp.concatenate([jnp.ones((NN, 1), f32), x, jnp.zeros((NN, SW - 2), f32)], axis=1)
    zer2 = jnp.zeros((ZROWS, SW), f32)
    zer32 = jnp.zeros((ZROWS, F), f32)

    dstp2 = dstp.reshape(EPAD // CH, CH)
    part = _scalar_pass(xs, srcp, dstp2, zer2)

    h, dinv = pl.pallas_call(
        _tc_stage0_body,
        grid=(GRID,),
        in_specs=[
            pl.BlockSpec((BN, 1), lambda i: (i, 0)),
            pl.BlockSpec((2, BN, SW), lambda i: (0, i, 0)),
            pl.BlockSpec((1, HID), lambda i: (0, 0)),
            pl.BlockSpec((1, HID), lambda i: (0, 0)),
            pl.BlockSpec((1, HID), lambda i: (0, 0)),
        ],
        out_specs=[
            pl.BlockSpec((2, BN, F), lambda i: (0, i, 0)),
            pl.BlockSpec((BN, 1), lambda i: (i, 0)),
        ],
        out_shape=[
            jax.ShapeDtypeStruct((2, NN, F), f32),
            jax.ShapeDtypeStruct((NN, 1), f32),
        ],
    )(x, part, Wl0.T, bl0.reshape(1, HID), Wr0.T)

    wmid = lambda Wl, bl, Wr: (Wl.T, bl.reshape(1, HID), Wr.T)

    agg1 = _agg_pass(h, srcp, dstp2, zer32)
    h = pl.pallas_call(
        _tc_mid_body,
        grid=(GRID,),
        in_specs=[
            pl.BlockSpec((2, BN, F), lambda i: (0, i, 0)),
            pl.BlockSpec((2, BN, F), lambda i: (0, i, 0)),
            pl.BlockSpec((BN, 1), lambda i: (i, 0)),
            pl.BlockSpec((HID, HID), lambda i: (0, 0)),
            pl.BlockSpec((1, HID), lambda i: (0, 0)),
            pl.BlockSpec((HID, HID), lambda i: (0, 0)),
        ],
        out_specs=pl.BlockSpec((2, BN, F), lambda i: (0, i, 0)),
        out_shape=jax.ShapeDtypeStruct((2, NN, F), f32),
    )(h, agg1, dinv, *wmid(Wl1, bl1, Wr1))

    agg2 = _agg_pass(h, srcp, dstp2, zer32)
    ex = jnp.array([[temperature, t]], dtype=f32)
    out = pl.pallas_call(
        _tc_final_body,
        grid=(GRID,),
        in_specs=[
            pl.BlockSpec((2, BN, F), lambda i: (0, i, 0)),
            pl.BlockSpec((2, BN, F), lambda i: (0, i, 0)),
            pl.BlockSpec((BN, 1), lambda i: (i, 0)),
            pl.BlockSpec((HID, HID), lambda i: (0, 0)),
            pl.BlockSpec((1, HID), lambda i: (0, 0)),
            pl.BlockSpec((HID, HID), lambda i: (0, 0)),
            pl.BlockSpec((HID, HID), lambda i: (0, 0)),
            pl.BlockSpec((1, HID), lambda i: (0, 0)),
            pl.BlockSpec((HID + 2, HID), lambda i: (0, 0)),
            pl.BlockSpec((1, HID), lambda i: (0, 0)),
            pl.BlockSpec((HID, 1), lambda i: (0, 0)),
            pl.BlockSpec((1, 1), lambda i: (0, 0)),
            pl.BlockSpec((1, 2), lambda i: (0, 0)),
        ],
        out_specs=pl.BlockSpec((1, 1), lambda i: (0, 0)),
        out_shape=jax.ShapeDtypeStruct((1, 1), f32),
        scratch_shapes=[pltpu.VMEM((1, HID), f32)],
    )(h, agg2, dinv, *wmid(Wl2, bl2, Wr2), Wv.T, bv.reshape(1, HID),
      W1.T, b1.reshape(1, HID), W2.T, b2.reshape(1, 1), ex)
    return out


# speedup vs baseline: 16.1598x; 1.1442x over previous
import jax
import jax.numpy as jnp
from jax import lax
from jax.experimental import pallas as pl
from jax.experimental.pallas import tpu as pltpu
from jax.experimental.pallas import tpu_sc as plsc

NN = 50000
HID = 64
F = 32
EE = 800000
CH = 128
NSUB = 16
NCORE = 2
EPAD = 811008
MAIN_CHUNKS = EPAD // (NSUB * CH)
HALF_CHUNKS = EPAD // (2 * NSUB * CH)
EHALF = EPAD // 2
NPAD = 51200
ZROWS = NPAD // NSUB
NOUT = 50048
OROWS = NOUT // NSUB
SW = 16
BN = 2000
GRID = NN // BN


def _make_sc_body(width, nbuf, chunks, scalar_mode):
    rounds = chunks // nbuf
    assert rounds % 2 == 0

    def body(table, srcp, dstp2, zer, out, sidx, didx, rows, acc, isem, gsem):
        c = lax.axis_index("c")
        s = lax.axis_index("s")
        if scalar_mode:
            chunk_base = c * (EPAD // (2 * CH)) + s * chunks
            tab = table
        else:
            chunk_base = s * chunks
            tab = table.at[c]
        def issue_idx(r, h):
            row0 = chunk_base + r * nbuf
            pltpu.async_copy(srcp.at[pl.ds(row0 * CH, nbuf * CH)],
                             sidx.at[h], isem.at[h])
            pltpu.async_copy(dstp2.at[pl.ds(row0, nbuf)],
                             didx.at[h], isem.at[h])

        def wait_idx(h):
            pltpu.make_async_copy(srcp.at[pl.ds(0, nbuf * CH)],
                                  sidx.at[h], isem.at[h]).wait()
            pltpu.make_async_copy(dstp2.at[pl.ds(0, nbuf)],
                                  didx.at[h], isem.at[h]).wait()

        def issue_gathers(h):
            for b in range(nbuf):
                pltpu.async_copy(tab.at[sidx.at[h, pl.ds(b * CH, CH)]],
                                 rows.at[h, b], gsem.at[h])

        def wait_gathers(h):
            for b in range(nbuf):
                pltpu.make_async_copy(tab.at[sidx.at[h, pl.ds(b * CH, CH)]],
                                      rows.at[h, b], gsem.at[h]).wait()

        def scatters(h):
            for b in range(nbuf):
                pltpu.sync_copy(rows.at[h, b], acc.at[didx.at[h, b]], add=True)

        def round_body(r, h, do_next, do_prefetch):
            if do_next:
                wait_idx(1 - h)
                issue_gathers(1 - h)
            wait_gathers(h)
            scatters(h)
            if do_prefetch:
                issue_idx(r + 2, h)

        issue_idx(0, 0)
        issue_idx(1, 1)
        wait_idx(0)
        issue_gathers(0)
        pltpu.sync_copy(zer, acc.at[pl.ds(s * ZROWS, ZROWS)])
        plsc.subcore_barrier()

        def pair(k2, carry):
            round_body(2 * k2, 0, True, True)
            round_body(2 * k2 + 1, 1, True, True)
            return carry

        lax.fori_loop(0, (rounds - 2) // 2, pair, 0)
        round_body(rounds - 2, 0, True, False)
        round_body(rounds - 1, 1, False, False)

        plsc.subcore_barrier()
        pltpu.sync_copy(acc.at[pl.ds(s * OROWS, OROWS)],
                        out.at[c, pl.ds(s * OROWS, OROWS)])

    return body, rounds


def _sc_mesh():
    return plsc.VectorSubcoreMesh(core_axis_name="c", subcore_axis_name="s",
                                  num_cores=NCORE, num_subcores=NSUB)


def _scalar_pass(xs, srcp, dstp2, zer):
    nbuf = 3
    body, _ = _make_sc_body(SW, nbuf, HALF_CHUNKS, True)
    return pl.kernel(
        body,
        out_type=jax.ShapeDtypeStruct((NCORE, NOUT, SW), jnp.float32),
        mesh=_sc_mesh(),
        scratch_types=[
            pltpu.VMEM((2, nbuf * CH), jnp.int32),
            pltpu.VMEM((2, nbuf, CH), jnp.int32),
            pltpu.VMEM((2, nbuf, CH, SW), jnp.float32),
            pltpu.VMEM_SHARED((NPAD, SW), jnp.float32),
            pltpu.SemaphoreType.DMA((2,)),
            pltpu.SemaphoreType.DMA((2,)),
        ],
        compiler_params=pltpu.CompilerParams(use_tc_tiling_on_sc=False),
    )(xs, srcp, dstp2, zer)


def _agg_pass(h2, srcp, dstp2, zer):
    nbuf = 3
    body, _ = _make_sc_body(F, nbuf, MAIN_CHUNKS, False)
    return pl.kernel(
        body,
        out_type=jax.ShapeDtypeStruct((NCORE, NOUT, F), jnp.float32),
        mesh=_sc_mesh(),
        scratch_types=[
            pltpu.VMEM((2, nbuf * CH), jnp.int32),
            pltpu.VMEM((2, nbuf, CH), jnp.int32),
            pltpu.VMEM((2, nbuf, CH, F), jnp.float32),
            pltpu.VMEM_SHARED((NPAD, F), jnp.float32),
            pltpu.SemaphoreType.DMA((2,)),
            pltpu.SemaphoreType.DMA((2,)),
        ],
        compiler_params=pltpu.CompilerParams(use_tc_tiling_on_sc=False),
    )(h2, srcp, dstp2, zer)


def _tc_stage0_body(x, p, wl, bl, wr, h_out, dinv_out):
    pb = p[...]
    deg = pb[0, :, 0:1] + pb[1, :, 0:1]
    di = 1.0 / jnp.maximum(deg, 1.0)
    a0 = (pb[0, :, 1:2] + pb[1, :, 1:2]) * di
    h1 = jnp.maximum(a0 * wl[...] + bl[...] + x[...] * wr[...], 0.0)
    dinv_out[...] = di
    h_out[0] = h1[:, :F]
    h_out[1] = h1[:, F:]


def _tc_mid_body(h, agg, dinv, wlT, bl, wrT, out):
    hb = jnp.concatenate([h[0], h[1]], axis=1)
    ab = jnp.concatenate([agg[0], agg[1]], axis=1) * dinv[...]
    hn = jnp.maximum(
        jnp.dot(ab, wlT[...], preferred_element_type=jnp.float32, precision=lax.Precision.HIGHEST) + bl[...]
        + jnp.dot(hb, wrT[...], preferred_element_type=jnp.float32, precision=lax.Precision.HIGHEST), 0.0)
    out[0] = hn[:, :F]
    out[1] = hn[:, F:]


def _tc_final_body(h, agg, dinv, wlT, bl, wrT, wvT, bv, w1T, b1, w2T, b2, ex,
                   out, acc):
    i = pl.program_id(0)

    @pl.when(i == 0)
    def _():
        acc[...] = jnp.zeros_like(acc)

    hb = jnp.concatenate([h[0], h[1]], axis=1)
    ab = jnp.concatenate([agg[0], agg[1]], axis=1) * dinv[...]
    h3 = jnp.maximum(
        jnp.dot(ab, wlT[...], preferred_element_type=jnp.float32, precision=lax.Precision.HIGHEST) + bl[...]
        + jnp.dot(hb, wrT[...], preferred_element_type=jnp.float32, precision=lax.Precision.HIGHEST), 0.0)
    acc[...] += jnp.sum(h3, axis=0, keepdims=True)

    @pl.when(i == GRID - 1)
    def _():
        pooled = acc[...] * (1.0 / NN)
        z = jnp.dot(pooled, wvT[...], preferred_element_type=jnp.float32, precision=lax.Precision.HIGHEST) + bv[...]
        feat = jnp.concatenate([z, ex[...]], axis=1)
        hm = jnp.maximum(
            jnp.dot(feat, w1T[...], preferred_element_type=jnp.float32, precision=lax.Precision.HIGHEST) + b1[...], 0.0)
        oo = jnp.dot(hm, w2T[...], preferred_element_type=jnp.float32, precision=lax.Precision.HIGHEST) + b2[...]
        out[...] = jnp.maximum(oo, 0.0) + jnp.log1p(jnp.exp(-jnp.abs(oo)))


def _full(i):
    return (0,) * i


def kernel(node_feats, edge_index, batch, temperature, t,
           Wl0, bl0, Wr0, Wl1, bl1, Wr1, Wl2, bl2, Wr2,
           Wv, bv, W1, b1, W2, b2):
    f32 = jnp.float32
    src = edge_index[0]
    dst = edge_index[1]
    pad = EPAD - EE
    ar = jnp.arange(pad, dtype=jnp.int32)
    srcp = jnp.concatenate([src, ar % NN])
    dstp = jnp.concatenate([dst, NN + (ar % (NPAD - NN))])
    x = node_feats.astype(f32)
    xs = jnp.concatenate([jnp.ones((NN, 1), f32), x, jnp.zeros((NN, SW - 2), f32)], axis=1)
    zer2 = jnp.zeros((ZROWS, SW), f32)
    zer32 = jnp.zeros((ZROWS, F), f32)

    dstp2 = dstp.reshape(EPAD // CH, CH)
    part = _scalar_pass(xs, srcp, dstp2, zer2)

    h, dinv = pl.pallas_call(
        _tc_stage0_body,
        grid=(GRID,),
        in_specs=[
            pl.BlockSpec((BN, 1), lambda i: (i, 0)),
            pl.BlockSpec((2, BN, SW), lambda i: (0, i, 0)),
            pl.BlockSpec((1, HID), lambda i: (0, 0)),
            pl.BlockSpec((1, HID), lambda i: (0, 0)),
            pl.BlockSpec((1, HID), lambda i: (0, 0)),
        ],
        out_specs=[
            pl.BlockSpec((2, BN, F), lambda i: (0, i, 0)),
            pl.BlockSpec((BN, 1), lambda i: (i, 0)),
        ],
        out_shape=[
            jax.ShapeDtypeStruct((2, NN, F), f32),
            jax.ShapeDtypeStruct((NN, 1), f32),
        ],
    )(x, part, Wl0.T, bl0.reshape(1, HID), Wr0.T)

    wmid = lambda Wl, bl, Wr: (Wl.T, bl.reshape(1, HID), Wr.T)

    agg1 = _agg_pass(h, srcp, dstp2, zer32)
    h = pl.pallas_call(
        _tc_mid_body,
        grid=(GRID,),
        in_specs=[
            pl.BlockSpec((2, BN, F), lambda i: (0, i, 0)),
            pl.BlockSpec((2, BN, F), lambda i: (0, i, 0)),
            pl.BlockSpec((BN, 1), lambda i: (i, 0)),
            pl.BlockSpec((HID, HID), lambda i: (0, 0)),
            pl.BlockSpec((1, HID), lambda i: (0, 0)),
            pl.BlockSpec((HID, HID), lambda i: (0, 0)),
        ],
        out_specs=pl.BlockSpec((2, BN, F), lambda i: (0, i, 0)),
        out_shape=jax.ShapeDtypeStruct((2, NN, F), f32),
    )(h, agg1, dinv, *wmid(Wl1, bl1, Wr1))

    agg2 = _agg_pass(h, srcp, dstp2, zer32)
    ex = jnp.array([[temperature, t]], dtype=f32)
    out = pl.pallas_call(
        _tc_final_body,
        grid=(GRID,),
        in_specs=[
            pl.BlockSpec((2, BN, F), lambda i: (0, i, 0)),
            pl.BlockSpec((2, BN, F), lambda i: (0, i, 0)),
            pl.BlockSpec((BN, 1), lambda i: (i, 0)),
            pl.BlockSpec((HID, HID), lambda i: (0, 0)),
            pl.BlockSpec((1, HID), lambda i: (0, 0)),
            pl.BlockSpec((HID, HID), lambda i: (0, 0)),
            pl.BlockSpec((HID, HID), lambda i: (0, 0)),
            pl.BlockSpec((1, HID), lambda i: (0, 0)),
            pl.BlockSpec((HID + 2, HID), lambda i: (0, 0)),
            pl.BlockSpec((1, HID), lambda i: (0, 0)),
            pl.BlockSpec((HID, 1), lambda i: (0, 0)),
            pl.BlockSpec((1, 1), lambda i: (0, 0)),
            pl.BlockSpec((1, 2), lambda i: (0, 0)),
        ],
        out_specs=pl.BlockSpec((1, 1), lambda i: (0, 0)),
        out_shape=jax.ShapeDtypeStruct((1, 1), f32),
        scratch_shapes=[pltpu.VMEM((1, HID), f32)],
    )(h, agg2, dinv, *wmid(Wl2, bl2, Wr2), Wv.T, bv.reshape(1, HID),
      W1.T, b1.reshape(1, HID), W2.T, b2.reshape(1, 1), ex)
    return out


# speedup vs baseline: 20.1967x; 1.2498x over previous
import jax
import jax.numpy as jnp
from jax import lax
from jax.experimental import pallas as pl
from jax.experimental.pallas import tpu as pltpu
from jax.experimental.pallas import tpu_sc as plsc

NN = 50000
HID = 64
F = 32
EE = 800000
CH = 128
NSUB = 16
NCORE = 2
EPAD = 811008
MAIN_CHUNKS = EPAD // (NSUB * CH)
HALF_CHUNKS = EPAD // (2 * NSUB * CH)
NPAD = 51200
ZROWS = NPAD // NSUB
NOUT = 50048
OROWS = NOUT // NSUB
BN = 2048
GRID = -(-NN // BN)
BH = BN * F // 128
NH1 = NN * F // 128
NOH = NOUT * F // 128


def _make_sc_body(nbuf, chunks, scalar_mode):
    rounds = chunks // nbuf
    assert rounds % 2 == 0

    def body(table, srcp, dstp2, zer, out, sidx, didx, rows, acc, isem, gsem):
        c = lax.axis_index("c")
        s = lax.axis_index("s")
        if scalar_mode:
            chunk_base = c * (EPAD // (2 * CH)) + s * chunks
            tab = table
        else:
            chunk_base = s * chunks
            tab = table.at[c]

        def issue_idx(r, h):
            row0 = chunk_base + r * nbuf
            pltpu.async_copy(srcp.at[pl.ds(row0 * CH, nbuf * CH)],
                             sidx.at[h], isem.at[h])
            pltpu.async_copy(dstp2.at[pl.ds(row0, nbuf)],
                             didx.at[h], isem.at[h])

        def wait_idx(h):
            pltpu.make_async_copy(srcp.at[pl.ds(0, nbuf * CH)],
                                  sidx.at[h], isem.at[h]).wait()
            pltpu.make_async_copy(dstp2.at[pl.ds(0, nbuf)],
                                  didx.at[h], isem.at[h]).wait()

        def issue_gathers(h):
            for b in range(nbuf):
                pltpu.async_copy(tab.at[sidx.at[h, pl.ds(b * CH, CH)]],
                                 rows.at[h, b], gsem.at[h])

        def wait_gathers(h):
            for b in range(nbuf):
                pltpu.make_async_copy(tab.at[sidx.at[h, pl.ds(b * CH, CH)]],
                                      rows.at[h, b], gsem.at[h]).wait()

        def scatters(h):
            for b in range(nbuf):
                pltpu.sync_copy(rows.at[h, b], acc.at[didx.at[h, b]], add=True)

        def round_body(r, h, do_next, do_prefetch):
            if do_next:
                wait_idx(1 - h)
                issue_gathers(1 - h)
            wait_gathers(h)
            scatters(h)
            if do_prefetch:
                issue_idx(r + 2, h)

        issue_idx(0, 0)
        issue_idx(1, 1)
        wait_idx(0)
        issue_gathers(0)
        pltpu.sync_copy(zer, acc.at[pl.ds(s * ZROWS, ZROWS)])
        plsc.subcore_barrier()

        def pair(k2, carry):
            round_body(2 * k2, 0, True, True)
            round_body(2 * k2 + 1, 1, True, True)
            return carry

        lax.fori_loop(0, (rounds - 2) // 2, pair, 0)
        round_body(rounds - 2, 0, True, False)
        round_body(rounds - 1, 1, False, False)

        plsc.subcore_barrier()
        pltpu.sync_copy(acc.at[pl.ds(s * OROWS, OROWS)],
                        out.at[c, pl.ds(s * OROWS, OROWS)])

    return body


def _sc_mesh():
    return plsc.VectorSubcoreMesh(core_axis_name="c", subcore_axis_name="s",
                                  num_cores=NCORE, num_subcores=NSUB)


def _sc_pass(table, srcp, dstp2, zer, nbuf, chunks, scalar_mode):
    return pl.kernel(
        _make_sc_body(nbuf, chunks, scalar_mode),
        out_type=jax.ShapeDtypeStruct((NCORE, NOUT, F), jnp.float32),
        mesh=_sc_mesh(),
        scratch_types=[
            pltpu.VMEM((2, nbuf * CH), jnp.int32),
            pltpu.VMEM((2, nbuf, CH), jnp.int32),
            pltpu.VMEM((2, nbuf, CH, F), jnp.float32),
            pltpu.VMEM_SHARED((NPAD, F), jnp.float32),
            pltpu.SemaphoreType.DMA((2,)),
            pltpu.SemaphoreType.DMA((2,)),
        ],
        compiler_params=pltpu.CompilerParams(use_tc_tiling_on_sc=False),
    )(table, srcp, dstp2, zer)


_HP = lax.Precision.HIGHEST


def _dt(a, w):
    return jnp.dot(a, w, preferred_element_type=jnp.float32, precision=_HP)


def _tc_stage0_body(x, p, s0, s1, wll, wlh, bll, blh, wrl, wrh,
                    h_out, di_out):
    ps = p[0] + p[1]
    lane = lax.broadcasted_iota(jnp.int32, (BH, 128), 1) % F
    di_s = jnp.where(lane == 0, 1.0 / jnp.maximum(ps, 1.0), 0.0)
    a0_s = jnp.where(lane == 1, ps, 0.0)
    di = _dt(di_s, s0[...])
    a0 = _dt(a0_s, s1[...]) * di
    xb = x[...]
    h_out[0] = jnp.maximum(a0 * wll[...] + bll[...] + xb * wrl[...], 0.0)
    h_out[1] = jnp.maximum(a0 * wlh[...] + blh[...] + xb * wrh[...], 0.0)
    di_out[...] = di


def _layer_z(h, agg, di, mats):
    all_, ahl, alh, ahh, rll, rhl, rlh, rhh, bll, blh = mats
    d = di[...]
    ab_lo = agg[0] * d
    ab_hi = agg[1] * d
    hb_lo = h[0]
    hb_hi = h[1]
    zlo = (_dt(ab_lo, all_[...]) + _dt(ab_hi, ahl[...])
           + _dt(hb_lo, rll[...]) + _dt(hb_hi, rhl[...]) + bll[...])
    zhi = (_dt(ab_lo, alh[...]) + _dt(ab_hi, ahh[...])
           + _dt(hb_lo, rlh[...]) + _dt(hb_hi, rhh[...]) + blh[...])
    return zlo, zhi


def _tc_mid_body(h, agg, di, all_, ahl, alh, ahh, rll, rhl, rlh, rhh,
                 bll, blh, out):
    zlo, zhi = _layer_z(h, agg, di,
                        (all_, ahl, alh, ahh, rll, rhl, rlh, rhh, bll, blh))
    out[0] = jnp.maximum(zlo, 0.0)
    out[1] = jnp.maximum(zhi, 0.0)


def _tc_final_body(h, agg, di, all_, ahl, alh, ahh, rll, rhl, rlh, rhh,
                   bll, blh, wvT, bv, w1T, b1, w2T, b2, ex, out, acc):
    i = pl.program_id(0)

    @pl.when(i == 0)
    def _():
        acc[...] = jnp.zeros_like(acc)

    zlo, zhi = _layer_z(h, agg, di,
                        (all_, ahl, alh, ahh, rll, rhl, rlh, rhh, bll, blh))
    h3lo = jnp.maximum(zlo, 0.0)
    h3hi = jnp.maximum(zhi, 0.0)
    row = lax.broadcasted_iota(jnp.int32, (BH, 128), 0)
    lane = lax.broadcasted_iota(jnp.int32, (BH, 128), 1)
    node = 4 * (i * BH + row) + lane // F
    m = node < NN
    h3lo = jnp.where(m, h3lo, 0.0)
    h3hi = jnp.where(m, h3hi, 0.0)
    slo = jnp.sum(h3lo, axis=0, keepdims=True)
    shi = jnp.sum(h3hi, axis=0, keepdims=True)
    acc[...] += jnp.concatenate([slo, shi], axis=0)

    @pl.when(i == GRID - 1)
    def _():
        a = acc[...]
        plo = (a[0:1, 0:32] + a[0:1, 32:64] + a[0:1, 64:96] + a[0:1, 96:128])
        phi = (a[1:2, 0:32] + a[1:2, 32:64] + a[1:2, 64:96] + a[1:2, 96:128])
        pooled = jnp.concatenate([plo, phi], axis=1) * (1.0 / NN)
        z = _dt(pooled, wvT[...]) + bv[...]
        feat = jnp.concatenate([z, ex[...]], axis=1)
        hm = jnp.maximum(_dt(feat, w1T[...]) + b1[...], 0.0)
        oo = _dt(hm, w2T[...]) + b2[...]
        out[...] = jnp.maximum(oo, 0.0) + jnp.log1p(jnp.exp(-jnp.abs(oo)))


def _row_spec():
    return pl.BlockSpec((BH, 128), lambda i: (i, 0))


def _pair_spec():
    return pl.BlockSpec((2, BH, 128), lambda i: (0, i, 0))


def _full_spec(shape):
    n = len(shape)
    return pl.BlockSpec(shape, lambda i: (0,) * n)


def kernel(node_feats, edge_index, batch, temperature, t,
           Wl0, bl0, Wr0, Wl1, bl1, Wr1, Wl2, bl2, Wr2,
           Wv, bv, W1, b1, W2, b2):
    f32 = jnp.float32
    src = edge_index[0]
    dst = edge_index[1]
    pad = EPAD - EE
    ar = jnp.arange(pad, dtype=jnp.int32)
    srcp = jnp.concatenate([src, ar % NN])
    dstp = jnp.concatenate([dst, NN + (ar % (NPAD - NN))])
    dstp2 = dstp.reshape(EPAD // CH, CH)

    x = node_feats.astype(f32)
    x128 = jnp.repeat(x.reshape(NH1, 4), F, axis=1)
    lanec = jnp.arange(128, dtype=jnp.int32) % F
    xs128 = jnp.where(lanec == 0, jnp.asarray(1.0, f32),
                      jnp.where(lanec == 1, x128, jnp.asarray(0.0, f32)))
    zer32 = jnp.zeros((ZROWS, F), f32)

    part = _sc_pass(xs128.reshape(NN, F), srcp, dstp2, zer32,
                    3, HALF_CHUNKS, True).reshape(2, NOH, 128)

    i4 = jnp.eye(4, dtype=f32)

    def kron4(b):
        return jnp.kron(i4, b)

    def wblocks(W):
        WT = W.T
        return (kron4(WT[:F, :F]), kron4(WT[F:, :F]),
                kron4(WT[:F, F:]), kron4(WT[F:, F:]))

    def tile128(v):
        return (jnp.tile(v[:F], 4).reshape(1, 128),
                jnp.tile(v[F:], 4).reshape(1, 128))

    s0 = kron4(jnp.zeros((F, F), f32).at[0, :].set(1.0))
    s1 = kron4(jnp.zeros((F, F), f32).at[1, :].set(1.0))
    wll0, wlh0 = tile128(Wl0[:, 0])
    wrl0, wrh0 = tile128(Wr0[:, 0])
    bll0, blh0 = tile128(bl0)

    h, di = pl.pallas_call(
        _tc_stage0_body,
        grid=(GRID,),
        in_specs=[
            _row_spec(), _pair_spec(),
            _full_spec((128, 128)), _full_spec((128, 128)),
            _full_spec((1, 128)), _full_spec((1, 128)),
            _full_spec((1, 128)), _full_spec((1, 128)),
            _full_spec((1, 128)), _full_spec((1, 128)),
        ],
        out_specs=[_pair_spec(), _row_spec()],
        out_shape=[
            jax.ShapeDtypeStruct((2, NH1, 128), f32),
            jax.ShapeDtypeStruct((NH1, 128), f32),
        ],
    )(x128, part, s0, s1, wll0, wlh0, bll0, blh0, wrl0, wrh0)

    def mid(hcur, Wl, bl, Wr, aggv):
        return pl.pallas_call(
            _tc_mid_body,
            grid=(GRID,),
            in_specs=[
                _pair_spec(), _pair_spec(), _row_spec(),
            ] + [_full_spec((128, 128))] * 8 + [
                _full_spec((1, 128)), _full_spec((1, 128)),
            ],
            out_specs=_pair_spec(),
            out_shape=jax.ShapeDtypeStruct((2, NH1, 128), f32),
        )(hcur, aggv, di, *wblocks(Wl), *wblocks(Wr), *tile128(bl))

    agg1 = _sc_pass(h.reshape(2, NN, F), srcp, dstp2, zer32,
                    3, MAIN_CHUNKS, False).reshape(2, NOH, 128)
    h = mid(h, Wl1, bl1, Wr1, agg1)

    agg2 = _sc_pass(h.reshape(2, NN, F), srcp, dstp2, zer32,
                    3, MAIN_CHUNKS, False).reshape(2, NOH, 128)
    ex = jnp.array([[temperature, t]], dtype=f32)
    out = pl.pallas_call(
        _tc_final_body,
        grid=(GRID,),
        in_specs=[
            _pair_spec(), _pair_spec(), _row_spec(),
        ] + [_full_spec((128, 128))] * 8 + [
            _full_spec((1, 128)), _full_spec((1, 128)),
            _full_spec((HID, HID)), _full_spec((1, HID)),
            _full_spec((HID + 2, HID)), _full_spec((1, HID)),
            _full_spec((HID, 1)), _full_spec((1, 1)),
            _full_spec((1, 2)),
        ],
        out_specs=pl.BlockSpec((1, 1), lambda i: (0, 0)),
        out_shape=jax.ShapeDtypeStruct((1, 1), f32),
        scratch_shapes=[pltpu.VMEM((2, 128), f32)],
    )(h, agg2, di, *wblocks(Wl2), *wblocks(Wr2), *tile128(bl2),
      Wv.T, bv.reshape(1, HID), W1.T, b1.reshape(1, HID),
      W2.T, b2.reshape(1, 1), ex)
    return out
